# baseline (speedup 1.0000x reference)
import functools

import jax
import jax.numpy as jnp
from jax import lax
from jax.experimental import pallas as pl
from jax.experimental.pallas import tpu as pltpu


def _transpose_body(w_ref, o_ref):
    o_ref[...] = w_ref[...].T.astype(o_ref.dtype)


def _transpose_cast(w, dtype):
    r, c = w.shape
    tr, tc = min(256, r), min(256, c)
    return pl.pallas_call(
        _transpose_body,
        out_shape=jax.ShapeDtypeStruct((c, r), dtype),
        grid=(r // tr, c // tc),
        in_specs=[pl.BlockSpec((tr, tc), lambda i, j: (i, j))],
        out_specs=pl.BlockSpec((tc, tr), lambda i, j: (j, i)),
    )(w)


def _lstm_body(x_ref, h0_ref, c0_ref, wih_ref, whh_ref, b_ref,
               wfc_ref, bfc_ref,
               out_ref, hn_ref, cn_ref,
               xw_sc, h_sc, c_sc,
               *, chunk_steps, batch_block, emb, hidden, out_classes,
               mm_dtype, unroll):
    ct, bb, E, H, O = chunk_steps, batch_block, emb, hidden, out_classes
    t = pl.program_id(0)

    @pl.when(t == 0)
    def _load_state():
        h_sc[...] = h0_ref[...]
        c_sc[...] = c0_ref[...]

    for s in range(ct):
        xs = x_ref[:, s * E:(s + 1) * E].astype(mm_dtype)
        xw_sc[s * bb:(s + 1) * bb, :] = (
            jnp.dot(xs, wih_ref[...], preferred_element_type=jnp.float32)
            + b_ref[...]
        ).astype(jnp.bfloat16)

    whh = whh_ref[...]

    def step(s, carry):
        h, c = carry
        row = pl.multiple_of(s * bb, bb)
        gates = xw_sc[pl.ds(row, bb), :].astype(jnp.float32) + jnp.dot(
            h.astype(mm_dtype), whh,
            preferred_element_type=jnp.float32)
        i_g = jax.nn.sigmoid(gates[:, 0:H])
        f_g = jax.nn.sigmoid(gates[:, H:2 * H])
        g_g = jnp.tanh(gates[:, 2 * H:3 * H])
        o_g = jax.nn.sigmoid(gates[:, 3 * H:4 * H])
        c_new = f_g * c + i_g * g_g
        h_new = o_g * jnp.tanh(c_new)
        return h_new, c_new

    h_T, c_T = lax.fori_loop(0, ct, step, (h_sc[...], c_sc[...]),
                             unroll=unroll)
    h_sc[...] = h_T
    c_sc[...] = c_T

    @pl.when(t == pl.num_programs(0) - 1)
    def _head():
        logits = (jnp.dot(h_T.astype(mm_dtype), wfc_ref[...],
                          preferred_element_type=jnp.float32)
                  + bfc_ref[...])
        m = jnp.max(logits, axis=-1, keepdims=True)
        lse = jnp.log(jnp.sum(jnp.exp(logits - m), axis=-1, keepdims=True)) + m
        out_ref[...] = logits - lse
        hn_ref[...] = h_T
        cn_ref[...] = c_T


def kernel(x, h0, c0, w_ih, w_hh, b_ih, b_hh, w_fc, b_fc):
    mm_dtype = jnp.bfloat16
    B, T, E = x.shape
    H = w_hh.shape[1]
    O = w_fc.shape[0]
    bb = B

    ct = T
    while ct * bb * 4 * H * 2 > 18 * 2**20:
        ct //= 2
    nt = T // ct
    unroll = min(8, ct)

    x2 = x.reshape(B, T * E)

    h0f = h0[0].astype(jnp.float32)
    c0f = c0[0].astype(jnp.float32)

    wih_t = _transpose_cast(w_ih, mm_dtype)
    whh_t = _transpose_cast(w_hh, mm_dtype)
    wfc_t = _transpose_cast(w_fc, mm_dtype)
    b = (b_ih + b_hh).reshape(1, 4 * H).astype(jnp.float32)
    bfc = b_fc.reshape(1, O).astype(jnp.float32)

    body = functools.partial(
        _lstm_body, chunk_steps=ct, batch_block=bb, emb=E, hidden=H,
        out_classes=O, mm_dtype=mm_dtype, unroll=unroll)

    nbytes = jnp.dtype(mm_dtype).itemsize
    flops = 2 * T * B * (E + H) * 4 * H + 2 * B * H * O
    transcendentals = T * B * 5 * H + B * O
    bytes_accessed = (x2.size * 4
                      + (wih_t.size + whh_t.size + wfc_t.size) * nbytes
                      + (h0f.size + c0f.size + b.size + bfc.size) * 4
                      + B * (O + 2 * H) * 4)
    est_vmem = (2 * ct * bb * E * 4 + ct * bb * 4 * H * 2
                + (wih_t.size + whh_t.size + wfc_t.size) * nbytes
                + bb * 8 * H * 4 + (2 << 20))
    vmem_limit = int(min(96 * 2**20, max(48 * 2**20, est_vmem + (8 << 20))))

    out, hn, cn = pl.pallas_call(
        body,
        out_shape=[jax.ShapeDtypeStruct((B, O), jnp.float32),
                   jax.ShapeDtypeStruct((B, H), jnp.float32),
                   jax.ShapeDtypeStruct((B, H), jnp.float32)],
        grid=(nt,),
        in_specs=[
            pl.BlockSpec((bb, ct * E), lambda t: (0, t)),
            pl.BlockSpec((bb, H), lambda t: (0, 0)),
            pl.BlockSpec((bb, H), lambda t: (0, 0)),
            pl.BlockSpec((E, 4 * H), lambda t: (0, 0)),
            pl.BlockSpec((H, 4 * H), lambda t: (0, 0)),
            pl.BlockSpec((1, 4 * H), lambda t: (0, 0)),
            pl.BlockSpec((H, O), lambda t: (0, 0)),
            pl.BlockSpec((1, O), lambda t: (0, 0)),
        ],
        out_specs=[pl.BlockSpec((bb, O), lambda t: (0, 0)),
                   pl.BlockSpec((bb, H), lambda t: (0, 0)),
                   pl.BlockSpec((bb, H), lambda t: (0, 0))],
        scratch_shapes=[
            pltpu.VMEM((ct * bb, 4 * H), jnp.bfloat16),
            pltpu.VMEM((bb, H), jnp.float32),
            pltpu.VMEM((bb, H), jnp.float32),
        ],
        compiler_params=pltpu.CompilerParams(
            dimension_semantics=("arbitrary",),
            vmem_limit_bytes=vmem_limit),
        cost_estimate=pl.CostEstimate(flops=int(flops),
                                      transcendentals=int(transcendentals),
                                      bytes_accessed=int(bytes_accessed)),
    )(x2, h0f, c0f, wih_t, whh_t, b, wfc_t, bfc)

    return out, hn[None], cn[None]


# speedup vs baseline: 1.0497x; 1.0497x over previous
import functools

import jax
import jax.numpy as jnp
from jax import lax
from jax.experimental import pallas as pl
from jax.experimental.pallas import tpu as pltpu


def _prep_body(wih_ref, whh_ref, wfc_ref, oih_ref, ohh_ref, ofc_ref):
    oih_ref[...] = wih_ref[...].T.astype(oih_ref.dtype)
    ohh_ref[...] = whh_ref[...].T.astype(ohh_ref.dtype)
    ofc_ref[...] = wfc_ref[...].T.astype(ofc_ref.dtype)


def _prep_weights(w_ih, w_hh, w_fc, dtype):
    shapes = [jax.ShapeDtypeStruct((w.shape[1], w.shape[0]), dtype)
              for w in (w_ih, w_hh, w_fc)]
    return pl.pallas_call(_prep_body, out_shape=shapes)(w_ih, w_hh, w_fc)


def _lstm_body(x_ref, h0_ref, c0_ref, wih_ref, whh_ref, b_ref,
               wfc_ref, bfc_ref,
               out_ref, hn_ref, cn_ref,
               xw_sc, h_sc, c_sc,
               *, chunk_steps, batch_block, emb, hidden, out_classes,
               mm_dtype, unroll):
    ct, bb, E, H, O = chunk_steps, batch_block, emb, hidden, out_classes
    t = pl.program_id(0)

    @pl.when(t == 0)
    def _load_state():
        h_sc[...] = h0_ref[...]
        c_sc[...] = c0_ref[...]

    for s in range(ct):
        xs = x_ref[:, s * E:(s + 1) * E].astype(mm_dtype)
        xw_sc[s * bb:(s + 1) * bb, :] = (
            jnp.dot(xs, wih_ref[...], preferred_element_type=jnp.float32)
            + b_ref[...]
        ).astype(jnp.bfloat16)

    whh = whh_ref[...]

    def step(s, carry):
        h, c = carry
        row = pl.multiple_of(s * bb, bb)
        gates = xw_sc[pl.ds(row, bb), :].astype(jnp.float32) + jnp.dot(
            h.astype(mm_dtype), whh,
            preferred_element_type=jnp.float32)
        i_g = jax.nn.sigmoid(gates[:, 0:H])
        f_g = jax.nn.sigmoid(gates[:, H:2 * H])
        g_g = jnp.tanh(gates[:, 2 * H:3 * H])
        o_g = jax.nn.sigmoid(gates[:, 3 * H:4 * H])
        c_new = f_g * c + i_g * g_g
        h_new = o_g * jnp.tanh(c_new)
        return h_new, c_new

    h_T, c_T = lax.fori_loop(0, ct, step, (h_sc[...], c_sc[...]),
                             unroll=unroll)
    h_sc[...] = h_T
    c_sc[...] = c_T

    @pl.when(t == pl.num_programs(0) - 1)
    def _head():
        logits = (jnp.dot(h_T.astype(mm_dtype), wfc_ref[...],
                          preferred_element_type=jnp.float32)
                  + bfc_ref[...])
        m = jnp.max(logits, axis=-1, keepdims=True)
        lse = jnp.log(jnp.sum(jnp.exp(logits - m), axis=-1, keepdims=True)) + m
        out_ref[...] = logits - lse
        hn_ref[...] = h_T
        cn_ref[...] = c_T


def kernel(x, h0, c0, w_ih, w_hh, b_ih, b_hh, w_fc, b_fc):
    mm_dtype = jnp.bfloat16
    B, T, E = x.shape
    H = w_hh.shape[1]
    O = w_fc.shape[0]
    bb = B

    ct = T
    while ct * bb * 4 * H * 2 > 18 * 2**20:
        ct //= 2
    nt = T // ct
    unroll = min(8, ct)

    x2 = x.reshape(B, T * E)

    h0f = h0[0].astype(jnp.float32)
    c0f = c0[0].astype(jnp.float32)

    wih_t, whh_t, wfc_t = _prep_weights(w_ih, w_hh, w_fc, mm_dtype)
    b = (b_ih + b_hh).reshape(1, 4 * H).astype(jnp.float32)
    bfc = b_fc.reshape(1, O).astype(jnp.float32)

    body = functools.partial(
        _lstm_body, chunk_steps=ct, batch_block=bb, emb=E, hidden=H,
        out_classes=O, mm_dtype=mm_dtype, unroll=unroll)

    nbytes = jnp.dtype(mm_dtype).itemsize
    flops = 2 * T * B * (E + H) * 4 * H + 2 * B * H * O
    transcendentals = T * B * 5 * H + B * O
    bytes_accessed = (x2.size * 4
                      + (wih_t.size + whh_t.size + wfc_t.size) * nbytes
                      + (h0f.size + c0f.size + b.size + bfc.size) * 4
                      + B * (O + 2 * H) * 4)
    est_vmem = (2 * ct * bb * E * 4 + ct * bb * 4 * H * 2
                + (wih_t.size + whh_t.size + wfc_t.size) * nbytes
                + bb * 8 * H * 4 + (2 << 20))
    vmem_limit = int(min(96 * 2**20, max(48 * 2**20, est_vmem + (8 << 20))))

    out, hn, cn = pl.pallas_call(
        body,
        out_shape=[jax.ShapeDtypeStruct((B, O), jnp.float32),
                   jax.ShapeDtypeStruct((B, H), jnp.float32),
                   jax.ShapeDtypeStruct((B, H), jnp.float32)],
        grid=(nt,),
        in_specs=[
            pl.BlockSpec((bb, ct * E), lambda t: (0, t)),
            pl.BlockSpec((bb, H), lambda t: (0, 0)),
            pl.BlockSpec((bb, H), lambda t: (0, 0)),
            pl.BlockSpec((E, 4 * H), lambda t: (0, 0)),
            pl.BlockSpec((H, 4 * H), lambda t: (0, 0)),
            pl.BlockSpec((1, 4 * H), lambda t: (0, 0)),
            pl.BlockSpec((H, O), lambda t: (0, 0)),
            pl.BlockSpec((1, O), lambda t: (0, 0)),
        ],
        out_specs=[pl.BlockSpec((bb, O), lambda t: (0, 0)),
                   pl.BlockSpec((bb, H), lambda t: (0, 0)),
                   pl.BlockSpec((bb, H), lambda t: (0, 0))],
        scratch_shapes=[
            pltpu.VMEM((ct * bb, 4 * H), jnp.bfloat16),
            pltpu.VMEM((bb, H), jnp.float32),
            pltpu.VMEM((bb, H), jnp.float32),
        ],
        compiler_params=pltpu.CompilerParams(
            dimension_semantics=("arbitrary",),
            vmem_limit_bytes=vmem_limit),
        cost_estimate=pl.CostEstimate(flops=int(flops),
                                      transcendentals=int(transcendentals),
                                      bytes_accessed=int(bytes_accessed)),
    )(x2, h0f, c0f, wih_t, whh_t, b, wfc_t, bfc)

    return out, hn[None], cn[None]


# speedup vs baseline: 1.1100x; 1.0575x over previous
import functools

import jax
import jax.numpy as jnp
from jax import lax
from jax.experimental import pallas as pl
from jax.experimental.pallas import tpu as pltpu


def _prep_body(wih_ref, whh_ref, wfc_ref, oih_ref, ohh_ref, ofc_ref):
    oih_ref[...] = wih_ref[...].T.astype(oih_ref.dtype)
    ohh_ref[...] = whh_ref[...].T.astype(ohh_ref.dtype)
    ofc_ref[...] = wfc_ref[...].T.astype(ofc_ref.dtype)


def _prep_weights(w_ih, w_hh, w_fc, dtype):
    shapes = [jax.ShapeDtypeStruct((w.shape[1], w.shape[0]), dtype)
              for w in (w_ih, w_hh, w_fc)]
    return pl.pallas_call(_prep_body, out_shape=shapes)(w_ih, w_hh, w_fc)


def _lstm_body(x_ref, h0_ref, c0_ref, wih_ref, whh_ref, b_ref,
               wfc_ref, bfc_ref,
               out_ref, hn_ref, cn_ref,
               xw_sc, h_sc, c_sc,
               *, chunk_steps, batch_block, emb, hidden, out_classes,
               mm_dtype, unroll):
    ct, bb, E, H, O = chunk_steps, batch_block, emb, hidden, out_classes
    t = pl.program_id(0)

    @pl.when(t == 0)
    def _load_state():
        h_sc[...] = h0_ref[...]
        c_sc[...] = c0_ref[...]

    for s in range(ct):
        xs = x_ref[:, s, :].astype(mm_dtype)
        xw_sc[s * bb:(s + 1) * bb, :] = (
            jnp.dot(xs, wih_ref[...], preferred_element_type=jnp.float32)
            + b_ref[...]
        ).astype(jnp.bfloat16)

    whh = whh_ref[...]

    def step(s, carry):
        h, c = carry
        row = pl.multiple_of(s * bb, bb)
        gates = xw_sc[pl.ds(row, bb), :].astype(jnp.float32) + jnp.dot(
            h.astype(mm_dtype), whh,
            preferred_element_type=jnp.float32)
        i_g = jax.nn.sigmoid(gates[:, 0:H])
        f_g = jax.nn.sigmoid(gates[:, H:2 * H])
        g_g = jnp.tanh(gates[:, 2 * H:3 * H])
        o_g = jax.nn.sigmoid(gates[:, 3 * H:4 * H])
        c_new = f_g * c + i_g * g_g
        h_new = o_g * jnp.tanh(c_new)
        return h_new, c_new

    h_T, c_T = lax.fori_loop(0, ct, step, (h_sc[...], c_sc[...]),
                             unroll=unroll)
    h_sc[...] = h_T
    c_sc[...] = c_T

    @pl.when(t == pl.num_programs(0) - 1)
    def _head():
        logits = (jnp.dot(h_T.astype(mm_dtype), wfc_ref[...],
                          preferred_element_type=jnp.float32)
                  + bfc_ref[...])
        m = jnp.max(logits, axis=-1, keepdims=True)
        lse = jnp.log(jnp.sum(jnp.exp(logits - m), axis=-1, keepdims=True)) + m
        out_ref[...] = logits - lse
        hn_ref[...] = h_T
        cn_ref[...] = c_T


def kernel(x, h0, c0, w_ih, w_hh, b_ih, b_hh, w_fc, b_fc):
    mm_dtype = jnp.bfloat16
    B, T, E = x.shape
    H = w_hh.shape[1]
    O = w_fc.shape[0]
    bb = B

    ct = T
    while ct * bb * 4 * H * 2 > 18 * 2**20:
        ct //= 2
    nt = T // ct
    unroll = min(8, ct)


    h0f = h0[0].astype(jnp.float32)
    c0f = c0[0].astype(jnp.float32)

    wih_t, whh_t, wfc_t = _prep_weights(w_ih, w_hh, w_fc, mm_dtype)
    b = (b_ih + b_hh).reshape(1, 4 * H).astype(jnp.float32)
    bfc = b_fc.reshape(1, O).astype(jnp.float32)

    body = functools.partial(
        _lstm_body, chunk_steps=ct, batch_block=bb, emb=E, hidden=H,
        out_classes=O, mm_dtype=mm_dtype, unroll=unroll)

    nbytes = jnp.dtype(mm_dtype).itemsize
    flops = 2 * T * B * (E + H) * 4 * H + 2 * B * H * O
    transcendentals = T * B * 5 * H + B * O
    bytes_accessed = (x.size * 4
                      + (wih_t.size + whh_t.size + wfc_t.size) * nbytes
                      + (h0f.size + c0f.size + b.size + bfc.size) * 4
                      + B * (O + 2 * H) * 4)
    est_vmem = (2 * ct * bb * E * 4 + ct * bb * 4 * H * 2
                + (wih_t.size + whh_t.size + wfc_t.size) * nbytes
                + bb * 8 * H * 4 + (2 << 20))
    vmem_limit = int(min(96 * 2**20, max(48 * 2**20, est_vmem + (8 << 20))))

    out, hn, cn = pl.pallas_call(
        body,
        out_shape=[jax.ShapeDtypeStruct((B, O), jnp.float32),
                   jax.ShapeDtypeStruct((B, H), jnp.float32),
                   jax.ShapeDtypeStruct((B, H), jnp.float32)],
        grid=(nt,),
        in_specs=[
            pl.BlockSpec((bb, ct, E), lambda t: (0, t, 0)),
            pl.BlockSpec((bb, H), lambda t: (0, 0)),
            pl.BlockSpec((bb, H), lambda t: (0, 0)),
            pl.BlockSpec((E, 4 * H), lambda t: (0, 0)),
            pl.BlockSpec((H, 4 * H), lambda t: (0, 0)),
            pl.BlockSpec((1, 4 * H), lambda t: (0, 0)),
            pl.BlockSpec((H, O), lambda t: (0, 0)),
            pl.BlockSpec((1, O), lambda t: (0, 0)),
        ],
        out_specs=[pl.BlockSpec((bb, O), lambda t: (0, 0)),
                   pl.BlockSpec((bb, H), lambda t: (0, 0)),
                   pl.BlockSpec((bb, H), lambda t: (0, 0))],
        scratch_shapes=[
            pltpu.VMEM((ct * bb, 4 * H), jnp.bfloat16),
            pltpu.VMEM((bb, H), jnp.float32),
            pltpu.VMEM((bb, H), jnp.float32),
        ],
        compiler_params=pltpu.CompilerParams(
            dimension_semantics=("arbitrary",),
            vmem_limit_bytes=vmem_limit),
        cost_estimate=pl.CostEstimate(flops=int(flops),
                                      transcendentals=int(transcendentals),
                                      bytes_accessed=int(bytes_accessed)),
    )(x, h0f, c0f, wih_t, whh_t, b, wfc_t, bfc)

    return out, hn[None], cn[None]
